# baseline (speedup 1.0000x reference)
import functools

import jax
import jax.numpy as jnp
from jax import lax
from jax.experimental import pallas as pl
from jax.experimental.pallas import tpu as pltpu
from jax.experimental.pallas import tpu_sc as plsc


def _make_sc_gather(B, S, D, N, M):
    info = plsc.get_sparse_core_info()
    NC, NS, L = info.num_cores, info.num_subcores, info.num_lanes
    NW = NC * NS
    RPW = N // NW

    mesh = plsc.VectorSubcoreMesh(core_axis_name="c", subcore_axis_name="s")

    @functools.partial(
        pl.kernel,
        out_type=jax.ShapeDtypeStruct((B * N, D), jnp.float32),
        mesh=mesh,
        scratch_types=[
            pltpu.VMEM((RPW,), jnp.int32),
            pltpu.VMEM((RPW,), jnp.int32),
            pltpu.VMEM((RPW, D), jnp.float32),
            pltpu.VMEM((D,), jnp.float32),
            pltpu.SemaphoreType.DMA,
        ],
    )
    def sc_gather(in_hbm, idx_hbm, mst_hbm, out_hbm,
                  idx_v, gidx_v, rows_v, mst_v, sem):
        wid = lax.axis_index("s") * NC + lax.axis_index("c")
        base = wid * RPW
        pltpu.sync_copy(idx_hbm.at[pl.ds(base, RPW)], idx_v)
        pltpu.sync_copy(mst_hbm, mst_v)

        def per_batch(b, _):
            ivecs = []
            for c in range(RPW // L):
                v = idx_v[pl.ds(c * L, L)]
                ivecs.append(v)
                gidx_v[pl.ds(c * L, L)] = jnp.maximum(v - M, 0) + b * S
            pltpu.async_copy(in_hbm.at[gidx_v], rows_v, sem).wait()

            for c in range(RPW // L):
                for l in range(L):
                    @pl.when(ivecs[c][l] < M)
                    def _():
                        j = c * L + l
                        for k in range(D // L):
                            rows_v[j, pl.ds(k * L, L)] = mst_v[pl.ds(k * L, L)]

            pltpu.sync_copy(rows_v, out_hbm.at[pl.ds(b * N + base, RPW)])
            return 0

        lax.fori_loop(0, B, per_batch, 0)

    return sc_gather


def kernel(inputs, mask_indices, un_masked_indices, mst):
    B, S, D = inputs.shape
    M = mask_indices.shape[0]
    N = M + un_masked_indices.shape[0]
    idx = jnp.concatenate([mask_indices, un_masked_indices]).astype(jnp.int32)
    sc_gather = _make_sc_gather(B, S, D, N, M)
    out_flat = sc_gather(inputs.reshape(B * S, D), idx,
                         mst.reshape(D).astype(inputs.dtype))
    return out_flat.reshape(B, N, D)


# speedup vs baseline: 2.1246x; 2.1246x over previous
import functools

import jax
import jax.numpy as jnp
from jax import lax
from jax.experimental import pallas as pl
from jax.experimental.pallas import tpu as pltpu
from jax.experimental.pallas import tpu_sc as plsc


def _make_sc_gather(B, S, D, N, M):
    info = plsc.get_sparse_core_info()
    NC, NS, L = info.num_cores, info.num_subcores, info.num_lanes
    NW = NC * NS
    RPW = N // NW

    mesh = plsc.VectorSubcoreMesh(core_axis_name="c", subcore_axis_name="s")

    @functools.partial(
        pl.kernel,
        out_type=jax.ShapeDtypeStruct((B * N, D), jnp.float32),
        mesh=mesh,
        scratch_types=[
            pltpu.VMEM((RPW,), jnp.int32),
            pltpu.VMEM((RPW,), jnp.int32),
            pltpu.VMEM((RPW,), jnp.int32),
            pltpu.VMEM((RPW, D), jnp.float32),
            pltpu.VMEM((D,), jnp.float32),
            pltpu.SemaphoreType.DMA,
        ],
    )
    def sc_gather(in_hbm, idx_hbm, gid_hbm, mst_hbm, out_hbm,
                  idx_v, gbase_v, gidx_v, rows_v, mst_v, sem):
        wid = lax.axis_index("s") * NC + lax.axis_index("c")
        base = wid * RPW
        pltpu.sync_copy(idx_hbm.at[pl.ds(base, RPW)], idx_v)
        pltpu.sync_copy(gid_hbm.at[pl.ds(base, RPW)], gbase_v)
        pltpu.sync_copy(mst_hbm, mst_v)

        def per_batch(b, _):
            ivecs = []
            for c in range(RPW // L):
                sl = pl.ds(c * L, L)
                ivecs.append(idx_v[sl])
                gidx_v[sl] = gbase_v[sl] + b * S
            pltpu.async_copy(in_hbm.at[gidx_v], rows_v, sem).wait()

            for c in range(RPW // L):
                for l in range(L):
                    @pl.when(ivecs[c][l] < M)
                    def _():
                        j = c * L + l
                        for k in range(D // L):
                            rows_v[j, pl.ds(k * L, L)] = mst_v[pl.ds(k * L, L)]

            pltpu.sync_copy(rows_v, out_hbm.at[pl.ds(b * N + base, RPW)])
            return 0

        lax.fori_loop(0, B, per_batch, 0)

    return sc_gather


def kernel(inputs, mask_indices, un_masked_indices, mst):
    B, S, D = inputs.shape
    M = mask_indices.shape[0]
    N = M + un_masked_indices.shape[0]
    idx = jnp.concatenate([mask_indices, un_masked_indices]).astype(jnp.int32)
    spread = (jnp.arange(N, dtype=jnp.int32) * 37) % S
    gid = jnp.where(idx >= M, idx - M, spread)
    sc_gather = _make_sc_gather(B, S, D, N, M)
    out_flat = sc_gather(inputs.reshape(B * S, D), idx, gid,
                         mst.reshape(D).astype(inputs.dtype))
    return out_flat.reshape(B, N, D)
